# Initial kernel scaffold; baseline (speedup 1.0000x reference)
import jax
import jax.numpy as jnp
from jax.experimental import pallas as pl


def kernel(edge_index, edge_vals, embeds):
    raise NotImplementedError("write your pallas kernel here")


# speedup vs baseline: 4.3920x; 4.3920x over previous
import functools

import jax
import jax.numpy as jnp
from jax import lax
from jax.experimental import pallas as pl
from jax.experimental.pallas import tpu as pltpu
from jax.experimental.pallas import tpu_sc as plsc

_NUM_CORES = 2
_NUM_SUBCORES = 16
_LANES = 16
_CHUNK = 80
_ZROWS = 128


@functools.partial(jax.jit, static_argnames=())
def _sc_spmm(src, dst, vals, embeds):
    n_edges = src.shape[0]
    d = embeds.shape[1]
    n_nodes = ((embeds.shape[0] + _NUM_SUBCORES * _ZROWS - 1)
               // (_NUM_SUBCORES * _ZROWS)) * (_NUM_SUBCORES * _ZROWS)
    nw = _NUM_CORES * _NUM_SUBCORES
    e_per_w = n_edges // nw
    n_chunks = e_per_w // _CHUNK
    rows_per_tile = n_nodes // _NUM_SUBCORES
    assert e_per_w * nw == n_edges
    assert n_chunks * _CHUNK == e_per_w
    assert rows_per_tile * _NUM_SUBCORES == n_nodes
    assert rows_per_tile % _ZROWS == 0
    assert d % _LANES == 0

    mesh = plsc.VectorSubcoreMesh(core_axis_name="c", subcore_axis_name="s")

    @functools.partial(
        pl.kernel,
        out_type=jax.ShapeDtypeStruct((_NUM_CORES, n_nodes, d), jnp.float32),
        mesh=mesh,
        scratch_types=[
            pltpu.VMEM((_CHUNK,), jnp.int32),
            pltpu.VMEM((_CHUNK,), jnp.int32),
            pltpu.VMEM((_CHUNK,), jnp.float32),
            pltpu.SMEM((_CHUNK,), jnp.float32),
            pltpu.VMEM((_CHUNK, d), jnp.float32),
            pltpu.VMEM((_ZROWS, d), jnp.float32),
            pltpu.VMEM_SHARED((n_nodes, d), jnp.float32),
            pltpu.SemaphoreType.DMA,
        ],
    )
    def k(src_hbm, dst_hbm, vals_hbm, emb_hbm, out_hbm,
          sidx, didx, vstage, vv, rows, zblk, accum, sem):
        c = lax.axis_index("c")
        s = lax.axis_index("s")
        wid = s * _NUM_CORES + c

        zeros16 = jnp.zeros((_LANES,), jnp.float32)

        def zrow(i, _):
            for j in range(d // _LANES):
                zblk[i, pl.ds(j * _LANES, _LANES)] = zeros16
            return 0

        lax.fori_loop(0, _ZROWS, zrow, 0)
        tile_base = s * rows_per_tile
        for z in range(rows_per_tile // _ZROWS):
            pltpu.sync_copy(zblk, accum.at[pl.ds(tile_base + z * _ZROWS, _ZROWS)])
        plsc.subcore_barrier()

        def chunk_body(t, _):
            base = wid * e_per_w + t * _CHUNK
            pltpu.sync_copy(src_hbm.at[pl.ds(base, _CHUNK)], sidx)
            pltpu.sync_copy(dst_hbm.at[pl.ds(base, _CHUNK)], didx)
            pltpu.sync_copy(vals_hbm.at[pl.ds(base, _CHUNK)], vstage)
            pltpu.async_copy(emb_hbm.at[sidx], rows, sem).wait()

            def group_body(g, _):
                val16 = vstage[pl.ds(g * _LANES, _LANES)]
                for r in range(_LANES):
                    i = g * _LANES + r
                    val = val16[r]
                    for j in range(d // _LANES):
                        sl = pl.ds(j * _LANES, _LANES)
                        rows[i, sl] = rows[i, sl] * val
                return 0

            lax.fori_loop(0, _CHUNK // _LANES, group_body, 0)
            pltpu.sync_copy(rows, accum.at[didx], add=True)
            return 0

        lax.fori_loop(0, n_chunks, chunk_body, 0)
        plsc.subcore_barrier()

        pltpu.sync_copy(accum.at[pl.ds(tile_base, rows_per_tile)],
                        out_hbm.at[c, pl.ds(tile_base, rows_per_tile)])

    return k(src, dst, vals, embeds)


def _add_block(a_ref, b_ref, o_ref):
    o_ref[...] = a_ref[...] + b_ref[...]


@jax.jit
def _combine(a, b):
    n_nodes, d = a.shape
    blk = n_nodes // 16
    return pl.pallas_call(
        _add_block,
        out_shape=jax.ShapeDtypeStruct((n_nodes, d), jnp.float32),
        grid=(n_nodes // blk,),
        in_specs=[pl.BlockSpec((blk, d), lambda i: (i, 0)),
                  pl.BlockSpec((blk, d), lambda i: (i, 0))],
        out_specs=pl.BlockSpec((blk, d), lambda i: (i, 0)),
    )(a, b)


def kernel(edge_index, edge_vals, embeds):
    dst = edge_index[0].astype(jnp.int32)
    src = edge_index[1].astype(jnp.int32)
    vals = edge_vals.astype(jnp.float32)
    partials = _sc_spmm(src, dst, vals, embeds.astype(jnp.float32))
    out = _combine(partials[0], partials[1])
    return out[: embeds.shape[0]]


# speedup vs baseline: 12.1007x; 2.7552x over previous
import functools

import jax
import jax.numpy as jnp
from jax import lax
from jax.experimental import pallas as pl
from jax.experimental.pallas import tpu as pltpu
from jax.experimental.pallas import tpu_sc as plsc

_NUM_CORES = 2
_NUM_SUBCORES = 16
_LANES = 16
_CHUNK = 80
_NBUF = 4
_NSTG = 8


@jax.jit
def _sc_spmm(src, dst, vals, embeds):
    nw = _NUM_CORES * _NUM_SUBCORES
    n_edges = src.shape[0]
    d = embeds.shape[1]
    e_per_w = n_edges // nw
    n_chunks = e_per_w // _CHUNK
    assert n_chunks * _CHUNK * nw == n_edges
    assert n_chunks >= _NSTG
    n_nodes = ((embeds.shape[0] + _NUM_SUBCORES * _CHUNK - 1)
               // (_NUM_SUBCORES * _CHUNK)) * (_NUM_SUBCORES * _CHUNK)
    rows_per_tile = n_nodes // _NUM_SUBCORES
    assert rows_per_tile % _CHUNK == 0
    assert d % _LANES == 0

    mesh = plsc.VectorSubcoreMesh(core_axis_name="c", subcore_axis_name="s")

    @functools.partial(
        pl.kernel,
        out_type=jax.ShapeDtypeStruct((_NUM_CORES, n_nodes, d), jnp.float32),
        mesh=mesh,
        scratch_types=[
            pltpu.VMEM((_NSTG, _CHUNK), jnp.int32),
            pltpu.VMEM((_NSTG, _CHUNK), jnp.int32),
            pltpu.VMEM((_NSTG, _CHUNK), jnp.float32),
            pltpu.VMEM((_NBUF, _CHUNK, d), jnp.float32),
            pltpu.VMEM_SHARED((n_nodes, d), jnp.float32),
            pltpu.SemaphoreType.DMA((_NSTG,)),
            pltpu.SemaphoreType.DMA((_NBUF,)),
            pltpu.SemaphoreType.DMA((_NBUF,)),
        ],
    )
    def k(src_hbm, dst_hbm, vals_hbm, emb_hbm, out_hbm,
          sidx, didx, valb, rows, accum, stsem, gsem, ssem):
        c = lax.axis_index("c")
        s = lax.axis_index("s")
        wid = s * _NUM_CORES + c
        wbase = wid * e_per_w

        def st_descs(t, slot):
            return (
                pltpu.make_async_copy(
                    src_hbm.at[pl.ds(wbase + t * _CHUNK, _CHUNK)],
                    sidx.at[slot], stsem.at[slot]),
                pltpu.make_async_copy(
                    dst_hbm.at[pl.ds(wbase + t * _CHUNK, _CHUNK)],
                    didx.at[slot], stsem.at[slot]),
                pltpu.make_async_copy(
                    vals_hbm.at[pl.ds(wbase + t * _CHUNK, _CHUNK)],
                    valb.at[slot], stsem.at[slot]),
            )

        def st_start(t, slot):
            for cp in st_descs(t, slot):
                cp.start()

        def st_wait(t, slot):
            for cp in st_descs(t, slot):
                cp.wait()

        def g_desc(t, b):
            return pltpu.make_async_copy(
                emb_hbm.at[sidx.at[lax.rem(t, _NSTG)]], rows.at[b], gsem.at[b])

        def s_desc(t, b):
            return pltpu.make_async_copy(
                rows.at[b], accum.at[didx.at[lax.rem(t, _NSTG)]], ssem.at[b])

        for t in range(_NSTG - 1):
            st_start(t, t)

        zeros16 = jnp.zeros((_LANES,), jnp.float32)

        def zrow(i, carry):
            for j in range(d // _LANES):
                rows[0, i, pl.ds(j * _LANES, _LANES)] = zeros16
            return carry

        lax.fori_loop(0, _CHUNK, zrow, 0)
        tile_base = s * rows_per_tile
        for z in range(rows_per_tile // _CHUNK):
            pltpu.sync_copy(rows.at[0],
                            accum.at[pl.ds(tile_base + z * _CHUNK, _CHUNK)])
        plsc.subcore_barrier()

        for t in range(_NBUF - 1):
            st_wait(t, t)
            g_desc(t, t).start()

        def outer(o, carry):
            for b in range(_NBUF):
                tt = o * _NBUF + b

                @pl.when(tt < n_chunks)
                def _process():
                    slot8 = lax.rem(tt, _NSTG)
                    prev = (b + _NBUF - 1) % _NBUF
                    g_desc(tt, b).wait()

                    def group(g, gc):
                        val16 = valb[slot8, pl.ds(g * _LANES, _LANES)]
                        for r in range(_LANES):
                            val = val16[r]
                            i = g * _LANES + r
                            for j in range(d // _LANES):
                                sl = pl.ds(j * _LANES, _LANES)
                                rows[b, i, sl] = rows[b, i, sl] * val
                        return gc

                    lax.fori_loop(0, _CHUNK // _LANES, group, 0)

                    @pl.when(tt >= 1)
                    def _drain_prev():
                        s_desc(tt - 1, prev).wait()

                    @pl.when(tt + _NSTG - 1 < n_chunks)
                    def _stage_ahead():
                        st_start(tt + _NSTG - 1, lax.rem(tt + _NSTG - 1, _NSTG))

                    @pl.when(tt + _NBUF - 1 < n_chunks)
                    def _prefetch():
                        st_wait(tt + _NBUF - 1, lax.rem(tt + _NBUF - 1, _NSTG))
                        g_desc(tt + _NBUF - 1, prev).start()

                    pltpu.async_copy(rows.at[b], accum.at[didx.at[slot8]],
                                     ssem.at[b], add=True)
            return carry

        n_outer = (n_chunks + _NBUF - 1) // _NBUF
        lax.fori_loop(0, n_outer, outer, 0)
        s_desc(n_chunks - 1, (n_chunks - 1) % _NBUF).wait()

        plsc.subcore_barrier()
        pltpu.sync_copy(accum.at[pl.ds(tile_base, rows_per_tile)],
                        out_hbm.at[c, pl.ds(tile_base, rows_per_tile)])

    return k(src, dst, vals, embeds)


def _add_block(a_ref, b_ref, o_ref):
    o_ref[...] = a_ref[...] + b_ref[...]


@jax.jit
def _combine(a, b):
    n_nodes, d = a.shape
    blk = n_nodes // 16
    return pl.pallas_call(
        _add_block,
        out_shape=jax.ShapeDtypeStruct((n_nodes, d), jnp.float32),
        grid=(n_nodes // blk,),
        in_specs=[pl.BlockSpec((blk, d), lambda i: (i, 0)),
                  pl.BlockSpec((blk, d), lambda i: (i, 0))],
        out_specs=pl.BlockSpec((blk, d), lambda i: (i, 0)),
    )(a, b)


def kernel(edge_index, edge_vals, embeds):
    dst = edge_index[0].astype(jnp.int32)
    src = edge_index[1].astype(jnp.int32)
    vals = edge_vals.astype(jnp.float32)
    partials = _sc_spmm(src, dst, vals, embeds.astype(jnp.float32))
    out = _combine(partials[0], partials[1])
    return out[: embeds.shape[0]]


# speedup vs baseline: 12.8629x; 1.0630x over previous
import functools

import jax
import jax.numpy as jnp
from jax import lax
from jax.experimental import pallas as pl
from jax.experimental.pallas import tpu as pltpu
from jax.experimental.pallas import tpu_sc as plsc

_NUM_CORES = 2
_NUM_SUBCORES = 16
_LANES = 16
_CHUNK = 80
_NBUF = 4
_NSTG = 8


@jax.jit
def _sc_spmm(src, dst, vals, embeds):
    nw = _NUM_CORES * _NUM_SUBCORES
    n_edges = src.shape[0]
    d = embeds.shape[1]
    e_per_w = n_edges // nw
    n_chunks = e_per_w // _CHUNK
    assert n_chunks * _CHUNK * nw == n_edges
    assert n_chunks >= _NSTG
    n_nodes = embeds.shape[0]
    n_zblocks = n_nodes // _CHUNK
    assert n_zblocks * _CHUNK == n_nodes
    _ndump = 10
    dump_rows = n_nodes // _ndump
    assert dump_rows * _ndump == n_nodes and dump_rows % 8 == 0
    assert d % _LANES == 0

    mesh = plsc.VectorSubcoreMesh(core_axis_name="c", subcore_axis_name="s")

    @functools.partial(
        pl.kernel,
        out_type=jax.ShapeDtypeStruct((_NUM_CORES, n_nodes, d), jnp.float32),
        mesh=mesh,
        scratch_types=[
            pltpu.VMEM((_NSTG, _CHUNK), jnp.int32),
            pltpu.VMEM((_NSTG, _CHUNK), jnp.int32),
            pltpu.VMEM((_NSTG, _CHUNK), jnp.float32),
            pltpu.VMEM((_NBUF, _CHUNK, d), jnp.float32),
            pltpu.VMEM_SHARED((n_nodes, d), jnp.float32),
            pltpu.SemaphoreType.DMA((_NSTG,)),
            pltpu.SemaphoreType.DMA((_NBUF,)),
            pltpu.SemaphoreType.DMA((_NBUF,)),
        ],
    )
    def k(src_hbm, dst_hbm, vals_hbm, emb_hbm, out_hbm,
          sidx, didx, valb, rows, accum, stsem, gsem, ssem):
        c = lax.axis_index("c")
        s = lax.axis_index("s")
        wid = s * _NUM_CORES + c
        wbase = wid * e_per_w

        def st_descs(t, slot):
            return (
                pltpu.make_async_copy(
                    src_hbm.at[pl.ds(wbase + t * _CHUNK, _CHUNK)],
                    sidx.at[slot], stsem.at[slot]),
                pltpu.make_async_copy(
                    dst_hbm.at[pl.ds(wbase + t * _CHUNK, _CHUNK)],
                    didx.at[slot], stsem.at[slot]),
                pltpu.make_async_copy(
                    vals_hbm.at[pl.ds(wbase + t * _CHUNK, _CHUNK)],
                    valb.at[slot], stsem.at[slot]),
            )

        def st_start(t, slot):
            for cp in st_descs(t, slot):
                cp.start()

        def st_wait(t, slot):
            for cp in st_descs(t, slot):
                cp.wait()

        def g_desc(t, b):
            return pltpu.make_async_copy(
                emb_hbm.at[sidx.at[lax.rem(t, _NSTG)]], rows.at[b], gsem.at[b])

        def s_desc(t, b):
            return pltpu.make_async_copy(
                rows.at[b], accum.at[didx.at[lax.rem(t, _NSTG)]], ssem.at[b])

        for t in range(_NSTG - 1):
            st_start(t, t)

        zeros16 = jnp.zeros((_LANES,), jnp.float32)

        def zrow(i, carry):
            for j in range(d // _LANES):
                rows[0, i, pl.ds(j * _LANES, _LANES)] = zeros16
            return carry

        lax.fori_loop(0, _CHUNK, zrow, 0)
        for z in range((n_zblocks + _NUM_SUBCORES - 1) // _NUM_SUBCORES):
            zb = z * _NUM_SUBCORES + s

            @pl.when(zb < n_zblocks)
            def _zero_block():
                pltpu.sync_copy(rows.at[0], accum.at[pl.ds(zb * _CHUNK, _CHUNK)])

        plsc.subcore_barrier()

        for t in range(_NBUF - 1):
            st_wait(t, t)
            g_desc(t, t).start()

        def outer(o, carry):
            for b in range(_NBUF):
                tt = o * _NBUF + b

                @pl.when(tt < n_chunks)
                def _process():
                    slot8 = lax.rem(tt, _NSTG)
                    prev = (b + _NBUF - 1) % _NBUF
                    g_desc(tt, b).wait()

                    def group(g, gc):
                        val16 = valb[slot8, pl.ds(g * _LANES, _LANES)]
                        for r in range(_LANES):
                            val = val16[r]
                            i = g * _LANES + r
                            for j in range(d // _LANES):
                                sl = pl.ds(j * _LANES, _LANES)
                                rows[b, i, sl] = rows[b, i, sl] * val
                        return gc

                    lax.fori_loop(0, _CHUNK // _LANES, group, 0)

                    @pl.when(tt >= 1)
                    def _drain_prev():
                        s_desc(tt - 1, prev).wait()

                    @pl.when(tt + _NSTG - 1 < n_chunks)
                    def _stage_ahead():
                        st_start(tt + _NSTG - 1, lax.rem(tt + _NSTG - 1, _NSTG))

                    @pl.when(tt + _NBUF - 1 < n_chunks)
                    def _prefetch():
                        st_wait(tt + _NBUF - 1, lax.rem(tt + _NBUF - 1, _NSTG))
                        g_desc(tt + _NBUF - 1, prev).start()

                    pltpu.async_copy(rows.at[b], accum.at[didx.at[slot8]],
                                     ssem.at[b], add=True)
            return carry

        n_outer = (n_chunks + _NBUF - 1) // _NBUF
        lax.fori_loop(0, n_outer, outer, 0)
        s_desc(n_chunks - 1, (n_chunks - 1) % _NBUF).wait()

        plsc.subcore_barrier()

        @pl.when(s < _ndump)
        def _dump():
            pltpu.sync_copy(accum.at[pl.ds(s * dump_rows, dump_rows)],
                            out_hbm.at[c, pl.ds(s * dump_rows, dump_rows)])

    return k(src, dst, vals, embeds)


def _add_block(a_ref, b_ref, o_ref):
    o_ref[...] = a_ref[...] + b_ref[...]


@jax.jit
def _combine(a, b):
    n_nodes, d = a.shape
    return pl.pallas_call(
        _add_block,
        out_shape=jax.ShapeDtypeStruct((n_nodes, d), jnp.float32),
    )(a, b)


def kernel(edge_index, edge_vals, embeds):
    dst = edge_index[0].astype(jnp.int32)
    src = edge_index[1].astype(jnp.int32)
    vals = edge_vals.astype(jnp.float32)
    partials = _sc_spmm(src, dst, vals, embeds.astype(jnp.float32))
    return _combine(partials[0], partials[1])
